# baseline (speedup 1.0000x reference)
import jax
import jax.numpy as jnp
from jax.experimental import pallas as pl
from jax.experimental.pallas import tpu as pltpu

_CHUNKS = 1


def _relay_body(x_ref, o_ref, *scratch):
    bufs = scratch[:_CHUNKS]
    in_sems, out_sems = scratch[_CHUNKS], scratch[_CHUNKS + 1]
    rows = x_ref.shape[0]
    per = rows // _CHUNKS
    ins = [
        pltpu.make_async_copy(
            x_ref.at[pl.ds(k * per, per), :], bufs[k], in_sems.at[k]
        )
        for k in range(_CHUNKS)
    ]
    outs = [
        pltpu.make_async_copy(
            bufs[k], o_ref.at[pl.ds(k * per, per), :], out_sems.at[k]
        )
        for k in range(_CHUNKS)
    ]
    for cp in ins:
        cp.start()
    for k in range(_CHUNKS):
        ins[k].wait()
        outs[k].start()
    for cp in outs:
        cp.wait()


def kernel(images, features, proposals, gt_bboxes, gt_labels):
    n, c, h, w = images.shape
    x = images.reshape(n * c * h, w)
    rows = x.shape[0]
    per = rows // _CHUNKS
    out = pl.pallas_call(
        _relay_body,
        out_shape=jax.ShapeDtypeStruct(x.shape, x.dtype),
        in_specs=[pl.BlockSpec(memory_space=pl.ANY)],
        out_specs=pl.BlockSpec(memory_space=pl.ANY),
        scratch_shapes=[pltpu.VMEM((per, w), x.dtype) for _ in range(_CHUNKS)]
        + [
            pltpu.SemaphoreType.DMA((_CHUNKS,)),
            pltpu.SemaphoreType.DMA((_CHUNKS,)),
        ],
    )(x)
    return out.reshape(images.shape)


# speedup vs baseline: 1.1592x; 1.1592x over previous
import jax
import jax.numpy as jnp
from jax.experimental import pallas as pl
from jax.experimental.pallas import tpu as pltpu

_CHUNKS = 3


def _relay_body(x_ref, o_ref, *scratch):
    bufs = scratch[:_CHUNKS]
    in_sems, out_sems = scratch[_CHUNKS], scratch[_CHUNKS + 1]
    rows = x_ref.shape[0]
    per = rows // _CHUNKS
    ins = [
        pltpu.make_async_copy(
            x_ref.at[pl.ds(k * per, per), :], bufs[k], in_sems.at[k]
        )
        for k in range(_CHUNKS)
    ]
    outs = [
        pltpu.make_async_copy(
            bufs[k], o_ref.at[pl.ds(k * per, per), :], out_sems.at[k]
        )
        for k in range(_CHUNKS)
    ]
    for cp in ins:
        cp.start()
    for k in range(_CHUNKS):
        ins[k].wait()
        outs[k].start()
    for cp in outs:
        cp.wait()


def kernel(images, features, proposals, gt_bboxes, gt_labels):
    n, c, h, w = images.shape
    x = images.reshape(n * c * h, w)
    rows = x.shape[0]
    per = rows // _CHUNKS
    out = pl.pallas_call(
        _relay_body,
        out_shape=jax.ShapeDtypeStruct(x.shape, x.dtype),
        in_specs=[pl.BlockSpec(memory_space=pl.ANY)],
        out_specs=pl.BlockSpec(memory_space=pl.ANY),
        scratch_shapes=[pltpu.VMEM((per, w), x.dtype) for _ in range(_CHUNKS)]
        + [
            pltpu.SemaphoreType.DMA((_CHUNKS,)),
            pltpu.SemaphoreType.DMA((_CHUNKS,)),
        ],
    )(x)
    return out.reshape(images.shape)


# speedup vs baseline: 1.1761x; 1.0146x over previous
import jax
import jax.numpy as jnp
from jax.experimental import pallas as pl
from jax.experimental.pallas import tpu as pltpu

_CHUNKS = 2


def _relay_body(x_ref, o_ref, *scratch):
    bufs = scratch[:_CHUNKS]
    in_sems, out_sems = scratch[_CHUNKS], scratch[_CHUNKS + 1]
    rows = x_ref.shape[0]
    per = rows // _CHUNKS
    ins = [
        pltpu.make_async_copy(
            x_ref.at[pl.ds(k * per, per), :], bufs[k], in_sems.at[k]
        )
        for k in range(_CHUNKS)
    ]
    outs = [
        pltpu.make_async_copy(
            bufs[k], o_ref.at[pl.ds(k * per, per), :], out_sems.at[k]
        )
        for k in range(_CHUNKS)
    ]
    for cp in ins:
        cp.start()
    for k in range(_CHUNKS):
        ins[k].wait()
        outs[k].start()
    for cp in outs:
        cp.wait()


def kernel(images, features, proposals, gt_bboxes, gt_labels):
    n, c, h, w = images.shape
    x = images.reshape(n * c * h, w)
    rows = x.shape[0]
    per = rows // _CHUNKS
    out = pl.pallas_call(
        _relay_body,
        out_shape=jax.ShapeDtypeStruct(x.shape, x.dtype),
        in_specs=[pl.BlockSpec(memory_space=pl.ANY)],
        out_specs=pl.BlockSpec(memory_space=pl.ANY),
        scratch_shapes=[pltpu.VMEM((per, w), x.dtype) for _ in range(_CHUNKS)]
        + [
            pltpu.SemaphoreType.DMA((_CHUNKS,)),
            pltpu.SemaphoreType.DMA((_CHUNKS,)),
        ],
    )(x)
    return out.reshape(images.shape)
